# baseline (speedup 1.0000x reference)
import functools

import jax
import jax.numpy as jnp
from jax import lax
from jax.experimental import pallas as pl
from jax.experimental.pallas import tpu as pltpu
from jax.experimental.pallas import tpu_sc as plsc

N = 10000
DEG = 16
E = N * DEG
D = 128
H = 128
HS = 8

NB = 400
EB = NB * DEG
NBLK = N // NB


_SC_CHUNK = 64


def _sc_gather_body(a_hbm, xt_hbm, yt_hbm, zt_hbm, idx_hbm,
                    outa_hbm, outx_hbm, outy_hbm, outz_hbm,
                    idx_v, rows_a, bx, by, bz, xt, yt, zt, sem_a):
    info = plsc.get_sparse_core_info()
    nc = info.num_cores
    wid = lax.axis_index("s") * nc + lax.axis_index("c")
    nw = nc * info.num_subcores
    per_w = E // nw
    nch = (per_w + _SC_CHUNK - 1) // _SC_CHUNK
    last = per_w - _SC_CHUNK

    pltpu.sync_copy(xt_hbm, xt)
    pltpu.sync_copy(yt_hbm, yt)
    pltpu.sync_copy(zt_hbm, zt)

    def body(i, _):
        base = wid * per_w + jnp.minimum(i * _SC_CHUNK, last)
        pltpu.sync_copy(idx_hbm.at[pl.ds(base, _SC_CHUNK)], idx_v)
        cp_a = pltpu.make_async_copy(a_hbm.at[idx_v], rows_a, sem_a)
        cp_a.start()
        for s in range(_SC_CHUNK // 16):
            reg = idx_v[pl.ds(16 * s, 16)]
            bx[pl.ds(16 * s, 16)] = plsc.load_gather(xt, [reg])
            by[pl.ds(16 * s, 16)] = plsc.load_gather(yt, [reg])
            bz[pl.ds(16 * s, 16)] = plsc.load_gather(zt, [reg])
        cp_a.wait()
        pltpu.sync_copy(rows_a, outa_hbm.at[pl.ds(base, _SC_CHUNK)])
        pltpu.sync_copy(bx, outx_hbm.at[pl.ds(base, _SC_CHUNK)])
        pltpu.sync_copy(by, outy_hbm.at[pl.ds(base, _SC_CHUNK)])
        pltpu.sync_copy(bz, outz_hbm.at[pl.ds(base, _SC_CHUNK)])
        return 0

    lax.fori_loop(0, nch, body, 0)


def _sc_gather(a_tab, xt, yt, zt, src):
    mesh = plsc.VectorSubcoreMesh(core_axis_name="c", subcore_axis_name="s")
    fn = pl.kernel(
        _sc_gather_body,
        mesh=mesh,
        compiler_params=pltpu.CompilerParams(needs_layout_passes=False),
        out_type=[
            jax.ShapeDtypeStruct((E, D), jnp.float32),
            jax.ShapeDtypeStruct((E,), jnp.float32),
            jax.ShapeDtypeStruct((E,), jnp.float32),
            jax.ShapeDtypeStruct((E,), jnp.float32),
        ],
        scratch_types=[
            pltpu.VMEM((_SC_CHUNK,), jnp.int32),
            pltpu.VMEM((_SC_CHUNK, D), jnp.float32),
            pltpu.VMEM((_SC_CHUNK,), jnp.float32),
            pltpu.VMEM((_SC_CHUNK,), jnp.float32),
            pltpu.VMEM((_SC_CHUNK,), jnp.float32),
            pltpu.VMEM((N,), jnp.float32),
            pltpu.VMEM((N,), jnp.float32),
            pltpu.VMEM((N,), jnp.float32),
            pltpu.SemaphoreType.DMA,
        ],
    )
    return fn(a_tab, xt, yt, zt, src)


def _prep_body(feat_ref, wa_ref, wb_ref, a_ref, b_ref):
    f = feat_ref[...]
    a_ref[...] = jnp.dot(f, wa_ref[...], preferred_element_type=jnp.float32)
    b_ref[...] = jnp.dot(f, wb_ref[...], preferred_element_type=jnp.float32)


def _prep(feat, wa, wb):
    return pl.pallas_call(
        _prep_body,
        grid=(NBLK,),
        in_specs=[
            pl.BlockSpec((NB, D), lambda i: (i, 0)),
            pl.BlockSpec((D, H), lambda i: (0, 0)),
            pl.BlockSpec((D, H), lambda i: (0, 0)),
        ],
        out_specs=[
            pl.BlockSpec((NB, H), lambda i: (i, 0)),
            pl.BlockSpec((NB, H), lambda i: (i, 0)),
        ],
        out_shape=[
            jax.ShapeDtypeStruct((N, H), jnp.float32),
            jax.ShapeDtypeStruct((N, H), jnp.float32),
        ],
    )(feat, wa, wb)


def _delta2d(mails, cols):
    nb = mails[0].shape[0]
    eb = nb * DEG
    acc = jnp.zeros((eb, DEG), jnp.float32)
    for mail, col in zip(mails, cols):
        mrep = jnp.broadcast_to(mail[:, None, :], (nb, DEG, DEG))
        mrep = mrep.reshape(eb, DEG)
        d = mrep - col
        acc = acc + d * d
    return acc


def _dsum_body(cxm_ref, cym_ref, czm_ref, out_ref):
    tot = jnp.zeros((), jnp.float32)
    for ref in (cxm_ref, cym_ref, czm_ref):
        m = ref[...]
        tot += 2.0 * DEG * jnp.sum(m * m)
        rs = jnp.sum(m, axis=1)
        tot -= 2.0 * jnp.sum(rs * rs)
    blk = tot.reshape(1, 1)

    @pl.when(pl.program_id(0) == 0)
    def _():
        out_ref[...] = jnp.zeros_like(out_ref)

    out_ref[...] += blk


def _dsum(cxm, cym, czm):
    return pl.pallas_call(
        _dsum_body,
        grid=(NBLK,),
        in_specs=[pl.BlockSpec((NB, 16), lambda i: (i, 0))] * 3,
        out_specs=pl.BlockSpec((1, 1), lambda i: (0, 0)),
        out_shape=jax.ShapeDtypeStruct((1, 1), jnp.float32),
    )(cxm, cym, czm)


def _silu(x):
    return x / (1.0 + jnp.exp(-x))


def _lane_tree(x, op):
    w = x.shape[1]
    while w > HS:
        half = w // 2
        x = op(x[:, :half], x[:, half:])
        w = half
    return x


def _main_body(as_ref, cxm_ref, cym_ref, czm_ref, cxc_ref, cyc_ref, czc_ref,
               coord_ref, feat_ref, b_ref, s_ref,
               w1t_ref, b1t_ref, w2big_ref, b2t_ref,
               esw_ref, esb_ref, nsw_ref, nsb_ref,
               ew1h_ref, ew1s_ref, eb1_ref, ew2_ref, eb2_ref,
               cw1_ref, cb1_ref, cw2_ref, cb2_ref,
               nw1f_ref, nw1h_ref, nw1v_ref, nb1_ref, nw2_ref, nb2_ref,
               h_out_ref, x_out_ref):
    mails = [cxm_ref[...], cym_ref[...], czm_ref[...]]
    cols = [cxc_ref[...], cyc_ref[...], czc_ref[...]]
    coord = coord_ref[...]

    inv = 1.0 / (s_ref[0, 0] + 1.0)
    delta = _delta2d(mails, cols) * inv

    rows16 = lax.broadcasted_iota(jnp.int32, (DEG, D), 0)
    lanes = lax.broadcasted_iota(jnp.int32, (DEG, D), 1)
    exp_mat = (lanes // HS == rows16).astype(jnp.bfloat16)
    delta_b = jnp.dot(delta.astype(jnp.bfloat16), exp_mat,
                      preferred_element_type=jnp.float32)

    h1 = _silu(delta_b * w1t_ref[...] + b1t_ref[...])
    h2 = _silu(jnp.dot(h1.astype(jnp.bfloat16), w2big_ref[...],
                       preferred_element_type=jnp.float32) + b2t_ref[...])

    lanes128 = lax.broadcasted_iota(jnp.int32, (D, HS), 0)
    ch8 = lax.broadcasted_iota(jnp.int32, (D, HS), 1)
    sum_mat = (lanes128 % HS == ch8).astype(jnp.bfloat16)
    s1 = jnp.dot(h2.astype(jnp.bfloat16), sum_mat,
                 preferred_element_type=jnp.float32)
    sq1 = jnp.dot((h2 * h2).astype(jnp.bfloat16), sum_mat,
                  preferred_element_type=jnp.float32)
    mean1 = s1 * (1.0 / DEG)
    var1 = sq1 * (1.0 / DEG) - mean1 * mean1
    std1 = jnp.sqrt(jnp.maximum(var1, 0.0))
    mx1 = _lane_tree(h2, jnp.maximum)
    mn1 = _lane_tree(h2, jnp.minimum)

    esw = esw_ref[...]
    acc = jnp.dot(s1, esw[0:8], preferred_element_type=jnp.float32)
    acc += jnp.dot(mean1, esw[8:16], preferred_element_type=jnp.float32)
    acc += jnp.dot(mx1, esw[16:24], preferred_element_type=jnp.float32)
    acc += jnp.dot(mn1, esw[24:32], preferred_element_type=jnp.float32)
    acc += jnp.dot(std1, esw[32:40], preferred_element_type=jnp.float32)
    h_e_dx = _silu(acc + esb_ref[...])

    nb = NB
    hr3 = h_e_dx.reshape(nb, DEG, HS)
    s2 = hr3[:, 0, :]
    sq2 = s2 * s2
    mx2 = s2
    mn2 = s2
    for i in range(1, DEG):
        v = hr3[:, i, :]
        s2 = s2 + v
        sq2 = sq2 + v * v
        mx2 = jnp.maximum(mx2, v)
        mn2 = jnp.minimum(mn2, v)
    mean2 = s2 * (1.0 / DEG)
    var2 = sq2 * (1.0 / DEG) - mean2 * mean2
    std2 = jnp.sqrt(jnp.maximum(var2, 0.0))
    nsw = nsw_ref[...]
    acc2 = jnp.dot(s2, nsw[0:8], preferred_element_type=jnp.float32)
    acc2 += jnp.dot(mean2, nsw[8:16], preferred_element_type=jnp.float32)
    acc2 += jnp.dot(mx2, nsw[16:24], preferred_element_type=jnp.float32)
    acc2 += jnp.dot(mn2, nsw[24:32], preferred_element_type=jnp.float32)
    acc2 += jnp.dot(std2, nsw[32:40], preferred_element_type=jnp.float32)
    h_v_dx = _silu(acc2 + nsb_ref[...])

    b_rep = jnp.broadcast_to(b_ref[...][:, None, :], (nb, DEG, H))
    b_rep = b_rep.reshape(EB, H)
    sqd = jnp.zeros((EB, 1), jnp.float32)
    dks = []
    for k in range(3):
        ck = coord[:, k:k + 1]
        crep = jnp.broadcast_to(ck[:, None, :], (nb, DEG, 1)).reshape(EB, 1)
        dk = cols[k] - crep
        dks.append(dk)
        sqd = sqd + dk * dk
    z1 = as_ref[...] + b_rep
    z1 += jnp.dot(h_e_dx, ew1h_ref[...], preferred_element_type=jnp.float32)
    z1 += sqd * ew1s_ref[...] + eb1_ref[...]
    h1e = _silu(z1)
    h_e = _silu(jnp.dot(h1e.astype(jnp.bfloat16), ew2_ref[...],
                        preferred_element_type=jnp.float32) + eb2_ref[...])

    ch = _silu(jnp.dot(h_e.astype(jnp.bfloat16), cw1_ref[...],
                       preferred_element_type=jnp.float32) + cb1_ref[...])
    coef = jnp.dot(ch, cw2_ref[...],
                   preferred_element_type=jnp.float32) + cb2_ref[0, 0]
    xcols = []
    for k in range(3):
        xe3 = (dks[k] * coef).reshape(nb, DEG, 1)
        xa = xe3[:, 0, :]
        for i in range(1, DEG):
            xa = xa + xe3[:, i, :]
        xcols.append(xa)
    x_out_ref[...] = coord + jnp.concatenate(xcols, axis=1)

    he3 = h_e.reshape(nb, DEG, H)
    h_agg = he3[:, 0, :]
    for i in range(1, DEG):
        h_agg = h_agg + he3[:, i, :]
    z = jnp.dot(feat_ref[...].astype(jnp.bfloat16), nw1f_ref[...],
                preferred_element_type=jnp.float32)
    z += jnp.dot(h_agg.astype(jnp.bfloat16), nw1h_ref[...],
                 preferred_element_type=jnp.float32)
    z += jnp.dot(h_v_dx, nw1v_ref[...], preferred_element_type=jnp.float32)
    h_new = jnp.dot(_silu(z + nb1_ref[...]).astype(jnp.bfloat16), nw2_ref[...],
                    preferred_element_type=jnp.float32) + nb2_ref[...]
    h_out_ref[...] = h_new


def _full(x):
    return pl.BlockSpec(x, lambda i: tuple(0 for _ in x))


def _main(as_g, cxm, cym, czm, cxc, cyc, czc, coordinate, feat, b_tab,
          s_val, wp):
    in_specs = [
        pl.BlockSpec((EB, D), lambda i: (i, 0)),
        pl.BlockSpec((NB, 16), lambda i: (i, 0)),
        pl.BlockSpec((NB, 16), lambda i: (i, 0)),
        pl.BlockSpec((NB, 16), lambda i: (i, 0)),
        pl.BlockSpec((EB, 1), lambda i: (i, 0)),
        pl.BlockSpec((EB, 1), lambda i: (i, 0)),
        pl.BlockSpec((EB, 1), lambda i: (i, 0)),
        pl.BlockSpec((NB, 3), lambda i: (i, 0)),
        pl.BlockSpec((NB, D), lambda i: (i, 0)),
        pl.BlockSpec((NB, H), lambda i: (i, 0)),
        _full((1, 1)),
        _full((1, D)), _full((1, D)), _full((D, D)), _full((1, D)),
        _full((5 * HS, HS)), _full((1, HS)), _full((5 * HS, HS)), _full((1, HS)),
        _full((HS, H)), _full((1, H)), _full((1, H)), _full((H, H)), _full((1, H)),
        _full((H, H)), _full((1, H)), _full((H, 1)), _full((1, 1)),
        _full((D, H)), _full((H, H)), _full((HS, H)), _full((1, H)), _full((H, D)),
        _full((1, D)),
    ]
    out_specs = [
        pl.BlockSpec((NB, D), lambda i: (i, 0)),
        pl.BlockSpec((NB, 3), lambda i: (i, 0)),
    ]
    return pl.pallas_call(
        _main_body,
        grid=(NBLK,),
        in_specs=in_specs,
        out_specs=out_specs,
        out_shape=[
            jax.ShapeDtypeStruct((N, D), jnp.float32),
            jax.ShapeDtypeStruct((N, 3), jnp.float32),
        ],
    )(as_g, cxm, cym, czm, cxc, cyc, czc, coordinate, feat, b_tab,
      s_val, *wp)


def _weight_prep(p):
    bf = jnp.bfloat16
    w1t = jnp.tile(p['dW1'].reshape(1, HS), (1, DEG))
    b1t = jnp.tile(p['db1'].reshape(1, HS), (1, DEG))
    w2big = jnp.kron(jnp.eye(DEG, dtype=jnp.float32), p['dW2'])
    b2t = jnp.tile(p['db2'].reshape(1, HS), (1, DEG))
    return [
        w1t, b1t, w2big.astype(bf), b2t,
        p['esW'], p['esb'].reshape(1, HS), p['nsW'], p['nsb'].reshape(1, HS),
        p['eW1'][0:HS], p['eW1'][HS + 2 * D:HS + 2 * D + 1],
        p['eb1'].reshape(1, H), p['eW2'].astype(bf), p['eb2'].reshape(1, H),
        p['cW1'].astype(bf), p['cb1'].reshape(1, H), p['cW2'],
        p['cb2'].reshape(1, 1),
        p['nW1'][0:D].astype(bf), p['nW1'][D:D + H].astype(bf),
        p['nW1'][D + H:D + H + HS],
        p['nb1'].reshape(1, H), p['nW2'].astype(bf), p['nb2'].reshape(1, D),
    ]


@jax.jit
def kernel(feat, coordinate, edge_index, params):
    p = params
    src = edge_index[0].astype(jnp.int32)
    wa = p['eW1'][HS:HS + D]
    wb = p['eW1'][HS + D:HS + 2 * D]
    a_tab, b_tab = _prep(feat, wa, wb)
    as_g, cx, cy, cz = _sc_gather(
        a_tab, coordinate[:, 0], coordinate[:, 1], coordinate[:, 2], src)
    cxm = cx.reshape(N, DEG)
    cym = cy.reshape(N, DEG)
    czm = cz.reshape(N, DEG)
    s_val = _dsum(cxm, cym, czm)
    wp = _weight_prep(p)
    h_new, x_new = _main(as_g, cxm, cym, czm, cx.reshape(E, 1),
                         cy.reshape(E, 1), cz.reshape(E, 1),
                         coordinate, feat, b_tab, s_val, wp)
    return h_new, x_new


# speedup vs baseline: 1.4015x; 1.4015x over previous
import functools

import jax
import jax.numpy as jnp
from jax import lax
from jax.experimental import pallas as pl
from jax.experimental.pallas import tpu as pltpu
from jax.experimental.pallas import tpu_sc as plsc

N = 10000
DEG = 16
E = N * DEG
D = 128
H = 128
HS = 8

NB = 400
EB = NB * DEG
NBLK = N // NB


_SC_CHUNK = 64


def _sc_gather_body(a_hbm, xt_hbm, yt_hbm, zt_hbm, idx_hbm,
                    outa_hbm, outx_hbm, outy_hbm, outz_hbm,
                    idx_v, rows_a, bx, by, bz, xt, yt, zt, sem_a):
    info = plsc.get_sparse_core_info()
    nc = info.num_cores
    wid = lax.axis_index("s") * nc + lax.axis_index("c")
    nw = nc * info.num_subcores
    per_w = E // nw
    nch = (per_w + _SC_CHUNK - 1) // _SC_CHUNK
    last = per_w - _SC_CHUNK

    pltpu.sync_copy(xt_hbm, xt)
    pltpu.sync_copy(yt_hbm, yt)
    pltpu.sync_copy(zt_hbm, zt)

    def body(i, _):
        base = wid * per_w + jnp.minimum(i * _SC_CHUNK, last)
        pltpu.sync_copy(idx_hbm.at[pl.ds(base, _SC_CHUNK)], idx_v)
        cp_a = pltpu.make_async_copy(a_hbm.at[idx_v], rows_a, sem_a)
        cp_a.start()
        for s in range(_SC_CHUNK // 16):
            reg = idx_v[pl.ds(16 * s, 16)]
            bx[pl.ds(16 * s, 16)] = plsc.load_gather(xt, [reg])
            by[pl.ds(16 * s, 16)] = plsc.load_gather(yt, [reg])
            bz[pl.ds(16 * s, 16)] = plsc.load_gather(zt, [reg])
        cp_a.wait()
        pltpu.sync_copy(rows_a, outa_hbm.at[pl.ds(base, _SC_CHUNK)])
        pltpu.sync_copy(bx, outx_hbm.at[pl.ds(base, _SC_CHUNK)])
        pltpu.sync_copy(by, outy_hbm.at[pl.ds(base, _SC_CHUNK)])
        pltpu.sync_copy(bz, outz_hbm.at[pl.ds(base, _SC_CHUNK)])
        return 0

    lax.fori_loop(0, nch, body, 0)


def _sc_gather(a_tab, xt, yt, zt, src):
    mesh = plsc.VectorSubcoreMesh(core_axis_name="c", subcore_axis_name="s")
    fn = pl.kernel(
        _sc_gather_body,
        mesh=mesh,
        compiler_params=pltpu.CompilerParams(needs_layout_passes=False),
        out_type=[
            jax.ShapeDtypeStruct((E, D), jnp.float32),
            jax.ShapeDtypeStruct((E,), jnp.float32),
            jax.ShapeDtypeStruct((E,), jnp.float32),
            jax.ShapeDtypeStruct((E,), jnp.float32),
        ],
        scratch_types=[
            pltpu.VMEM((_SC_CHUNK,), jnp.int32),
            pltpu.VMEM((_SC_CHUNK, D), jnp.float32),
            pltpu.VMEM((_SC_CHUNK,), jnp.float32),
            pltpu.VMEM((_SC_CHUNK,), jnp.float32),
            pltpu.VMEM((_SC_CHUNK,), jnp.float32),
            pltpu.VMEM((N,), jnp.float32),
            pltpu.VMEM((N,), jnp.float32),
            pltpu.VMEM((N,), jnp.float32),
            pltpu.SemaphoreType.DMA,
        ],
    )
    return fn(a_tab, xt, yt, zt, src)


def _prep_body(feat_ref, wa_ref, wb_ref, a_ref, b_ref):
    f = feat_ref[...]
    a_ref[...] = jnp.dot(f, wa_ref[...], preferred_element_type=jnp.float32)
    b_ref[...] = jnp.dot(f, wb_ref[...], preferred_element_type=jnp.float32)


def _prep(feat, wa, wb):
    return pl.pallas_call(
        _prep_body,
        grid=(NBLK,),
        in_specs=[
            pl.BlockSpec((NB, D), lambda i: (i, 0)),
            pl.BlockSpec((D, H), lambda i: (0, 0)),
            pl.BlockSpec((D, H), lambda i: (0, 0)),
        ],
        out_specs=[
            pl.BlockSpec((NB, H), lambda i: (i, 0)),
            pl.BlockSpec((NB, H), lambda i: (i, 0)),
        ],
        out_shape=[
            jax.ShapeDtypeStruct((N, H), jnp.float32),
            jax.ShapeDtypeStruct((N, H), jnp.float32),
        ],
    )(feat, wa, wb)


def _delta2d(mails, cols):
    nb = mails[0].shape[0]
    eb = nb * DEG
    acc = jnp.zeros((eb, DEG), jnp.float32)
    for mail, col in zip(mails, cols):
        mrep = jnp.broadcast_to(mail[:, None, :], (nb, DEG, DEG))
        mrep = mrep.reshape(eb, DEG)
        d = mrep - col
        acc = acc + d * d
    return acc


def _dsum_body(cxm_ref, cym_ref, czm_ref, out_ref):
    tot = jnp.zeros((), jnp.float32)
    for ref in (cxm_ref, cym_ref, czm_ref):
        m = ref[...]
        tot += 2.0 * DEG * jnp.sum(m * m)
        rs = jnp.sum(m, axis=1)
        tot -= 2.0 * jnp.sum(rs * rs)
    blk = tot.reshape(1, 1)

    @pl.when(pl.program_id(0) == 0)
    def _():
        out_ref[...] = jnp.zeros_like(out_ref)

    out_ref[...] += blk


def _dsum(cxm, cym, czm):
    return pl.pallas_call(
        _dsum_body,
        grid=(NBLK,),
        in_specs=[pl.BlockSpec((NB, 16), lambda i: (i, 0))] * 3,
        out_specs=pl.BlockSpec((1, 1), lambda i: (0, 0)),
        out_shape=jax.ShapeDtypeStruct((1, 1), jnp.float32),
    )(cxm, cym, czm)


def _silu(x):
    return 0.5 * x * (jnp.tanh(0.5 * x) + 1.0)


def _lane_tree(x, op):
    w = x.shape[1]
    while w > HS:
        half = w // 2
        x = op(x[:, :half], x[:, half:])
        w = half
    return x


def _main_body(as_ref, cxm_ref, cym_ref, czm_ref, ccol_ref, crep_ref,
               coord_ref, feat_ref, b_ref, s_ref,
               w1t_ref, b1t_ref, w2big_ref, b2t_ref,
               esw_ref, esb_ref, nsw_ref, nsb_ref,
               ew1h_ref, ew1s_ref, eb1_ref, ew2_ref, eb2_ref,
               cw1_ref, cb1_ref, cw2_ref, cb2_ref,
               nw1f_ref, nw1h_ref, nw1v_ref, nb1_ref, nw2_ref, nb2_ref,
               h_out_ref, x_out_ref):
    mails = [cxm_ref[...], cym_ref[...], czm_ref[...]]
    ccol = ccol_ref[...]
    cols = [ccol[:, k:k + 1] for k in range(3)]
    coord = coord_ref[...]

    inv = 1.0 / (s_ref[0, 0] + 1.0)
    delta = _delta2d(mails, cols) * inv

    rows16 = lax.broadcasted_iota(jnp.int32, (DEG, D), 0)
    lanes = lax.broadcasted_iota(jnp.int32, (DEG, D), 1)
    exp_mat = (lanes // HS == rows16).astype(jnp.bfloat16)
    delta_b = jnp.dot(delta.astype(jnp.bfloat16), exp_mat,
                      preferred_element_type=jnp.float32)

    h1 = _silu(delta_b * w1t_ref[...] + b1t_ref[...])
    h2 = _silu(jnp.dot(h1.astype(jnp.bfloat16), w2big_ref[...],
                       preferred_element_type=jnp.float32) + b2t_ref[...])

    lanes128 = lax.broadcasted_iota(jnp.int32, (D, HS), 0)
    ch8 = lax.broadcasted_iota(jnp.int32, (D, HS), 1)
    sum_mat = (lanes128 % HS == ch8).astype(jnp.bfloat16)
    h2b = h2.astype(jnp.bfloat16)
    s1 = jnp.dot(h2b, sum_mat, preferred_element_type=jnp.float32)
    sq1 = jnp.dot(h2b * h2b, sum_mat, preferred_element_type=jnp.float32)
    mean1 = s1 * (1.0 / DEG)
    var1 = sq1 * (1.0 / DEG) - mean1 * mean1
    std1 = jnp.sqrt(jnp.maximum(var1, 0.0))
    mx1 = _lane_tree(h2b, jnp.maximum)
    mn1 = _lane_tree(h2b, jnp.minimum)

    esw = esw_ref[...]
    eswb = esw.astype(jnp.bfloat16)
    acc = jnp.dot(s1, esw[0:8], preferred_element_type=jnp.float32)
    acc += jnp.dot(mean1, esw[8:16], preferred_element_type=jnp.float32)
    acc += jnp.dot(mx1, eswb[16:24], preferred_element_type=jnp.float32)
    acc += jnp.dot(mn1, eswb[24:32], preferred_element_type=jnp.float32)
    acc += jnp.dot(std1, esw[32:40], preferred_element_type=jnp.float32)
    h_e_dx = _silu(acc + esb_ref[...])

    nb = NB
    hr3 = h_e_dx.reshape(nb, DEG, HS)
    s2 = hr3[:, 0, :]
    sq2 = s2 * s2
    mx2 = s2
    mn2 = s2
    for i in range(1, DEG):
        v = hr3[:, i, :]
        s2 = s2 + v
        sq2 = sq2 + v * v
        mx2 = jnp.maximum(mx2, v)
        mn2 = jnp.minimum(mn2, v)
    mean2 = s2 * (1.0 / DEG)
    var2 = sq2 * (1.0 / DEG) - mean2 * mean2
    std2 = jnp.sqrt(jnp.maximum(var2, 0.0))
    nsw = nsw_ref[...]
    acc2 = jnp.dot(s2, nsw[0:8], preferred_element_type=jnp.float32)
    acc2 += jnp.dot(mean2, nsw[8:16], preferred_element_type=jnp.float32)
    acc2 += jnp.dot(mx2, nsw[16:24], preferred_element_type=jnp.float32)
    acc2 += jnp.dot(mn2, nsw[24:32], preferred_element_type=jnp.float32)
    acc2 += jnp.dot(std2, nsw[32:40], preferred_element_type=jnp.float32)
    h_v_dx = _silu(acc2 + nsb_ref[...])

    b_rep = jnp.broadcast_to(b_ref[...][:, None, :], (nb, DEG, H))
    b_rep = b_rep.reshape(EB, H)
    dcat = ccol - crep_ref[...]
    sqd = jnp.sum(dcat * dcat, axis=1, keepdims=True)
    z1 = as_ref[...] + b_rep
    z1 += jnp.dot(h_e_dx, ew1h_ref[...], preferred_element_type=jnp.float32)
    z1 += sqd * ew1s_ref[...] + eb1_ref[...]
    h1e = _silu(z1)
    h_e = _silu(jnp.dot(h1e.astype(jnp.bfloat16), ew2_ref[...],
                        preferred_element_type=jnp.float32) + eb2_ref[...])

    ch = _silu(jnp.dot(h_e.astype(jnp.bfloat16), cw1_ref[...],
                       preferred_element_type=jnp.float32) + cb1_ref[...])
    coef = jnp.dot(ch, cw2_ref[...],
                   preferred_element_type=jnp.float32) + cb2_ref[0, 0]
    g3 = (dcat * coef).reshape(nb, DEG, 3)
    xa = g3[:, 0, :]
    for i in range(1, DEG):
        xa = xa + g3[:, i, :]
    x_out_ref[...] = coord + xa

    he3 = h_e.reshape(nb, DEG, H)
    h_agg = he3[:, 0, :]
    for i in range(1, DEG):
        h_agg = h_agg + he3[:, i, :]
    z = jnp.dot(feat_ref[...].astype(jnp.bfloat16), nw1f_ref[...],
                preferred_element_type=jnp.float32)
    z += jnp.dot(h_agg.astype(jnp.bfloat16), nw1h_ref[...],
                 preferred_element_type=jnp.float32)
    z += jnp.dot(h_v_dx, nw1v_ref[...], preferred_element_type=jnp.float32)
    h_new = jnp.dot(_silu(z + nb1_ref[...]).astype(jnp.bfloat16), nw2_ref[...],
                    preferred_element_type=jnp.float32) + nb2_ref[...]
    h_out_ref[...] = h_new


def _full(x):
    return pl.BlockSpec(x, lambda i: tuple(0 for _ in x))


def _main(as_g, cxm, cym, czm, ccol, crep, coordinate, feat, b_tab,
          s_val, wp):
    in_specs = [
        pl.BlockSpec((EB, D), lambda i: (i, 0)),
        pl.BlockSpec((NB, 16), lambda i: (i, 0)),
        pl.BlockSpec((NB, 16), lambda i: (i, 0)),
        pl.BlockSpec((NB, 16), lambda i: (i, 0)),
        pl.BlockSpec((EB, 3), lambda i: (i, 0)),
        pl.BlockSpec((EB, 3), lambda i: (i, 0)),
        pl.BlockSpec((NB, 3), lambda i: (i, 0)),
        pl.BlockSpec((NB, D), lambda i: (i, 0)),
        pl.BlockSpec((NB, H), lambda i: (i, 0)),
        _full((1, 1)),
        _full((1, D)), _full((1, D)), _full((D, D)), _full((1, D)),
        _full((5 * HS, HS)), _full((1, HS)), _full((5 * HS, HS)), _full((1, HS)),
        _full((HS, H)), _full((1, H)), _full((1, H)), _full((H, H)), _full((1, H)),
        _full((H, H)), _full((1, H)), _full((H, 1)), _full((1, 1)),
        _full((D, H)), _full((H, H)), _full((HS, H)), _full((1, H)), _full((H, D)),
        _full((1, D)),
    ]
    out_specs = [
        pl.BlockSpec((NB, D), lambda i: (i, 0)),
        pl.BlockSpec((NB, 3), lambda i: (i, 0)),
    ]
    return pl.pallas_call(
        _main_body,
        grid=(NBLK,),
        in_specs=in_specs,
        out_specs=out_specs,
        out_shape=[
            jax.ShapeDtypeStruct((N, D), jnp.float32),
            jax.ShapeDtypeStruct((N, 3), jnp.float32),
        ],
    )(as_g, cxm, cym, czm, ccol, crep, coordinate, feat, b_tab,
      s_val, *wp)


def _weight_prep(p):
    bf = jnp.bfloat16
    w1t = jnp.tile(p['dW1'].reshape(1, HS), (1, DEG))
    b1t = jnp.tile(p['db1'].reshape(1, HS), (1, DEG))
    w2big = jnp.kron(jnp.eye(DEG, dtype=jnp.float32), p['dW2'])
    b2t = jnp.tile(p['db2'].reshape(1, HS), (1, DEG))
    return [
        w1t, b1t, w2big.astype(bf), b2t,
        p['esW'], p['esb'].reshape(1, HS), p['nsW'], p['nsb'].reshape(1, HS),
        p['eW1'][0:HS], p['eW1'][HS + 2 * D:HS + 2 * D + 1],
        p['eb1'].reshape(1, H), p['eW2'].astype(bf), p['eb2'].reshape(1, H),
        p['cW1'].astype(bf), p['cb1'].reshape(1, H), p['cW2'],
        p['cb2'].reshape(1, 1),
        p['nW1'][0:D].astype(bf), p['nW1'][D:D + H].astype(bf),
        p['nW1'][D + H:D + H + HS],
        p['nb1'].reshape(1, H), p['nW2'].astype(bf), p['nb2'].reshape(1, D),
    ]


@jax.jit
def kernel(feat, coordinate, edge_index, params):
    p = params
    src = edge_index[0].astype(jnp.int32)
    wa = p['eW1'][HS:HS + D]
    wb = p['eW1'][HS + D:HS + 2 * D]
    a_tab, b_tab = _prep(feat, wa, wb)
    as_g, cx, cy, cz = _sc_gather(
        a_tab, coordinate[:, 0], coordinate[:, 1], coordinate[:, 2], src)
    cxm = cx.reshape(N, DEG)
    cym = cy.reshape(N, DEG)
    czm = cz.reshape(N, DEG)
    s_val = _dsum(cxm, cym, czm)
    wp = _weight_prep(p)
    ccol = jnp.stack([cx, cy, cz], axis=1)
    crep = jnp.repeat(coordinate, DEG, axis=0)
    h_new, x_new = _main(as_g, cxm, cym, czm, ccol, crep,
                         coordinate, feat, b_tab, s_val, wp)
    return h_new, x_new


# speedup vs baseline: 1.5300x; 1.0917x over previous
import functools

import jax
import jax.numpy as jnp
from jax import lax
from jax.experimental import pallas as pl
from jax.experimental.pallas import tpu as pltpu
from jax.experimental.pallas import tpu_sc as plsc

N = 10000
DEG = 16
E = N * DEG
D = 128
H = 128
HS = 8

NB = 400
EB = NB * DEG
NBLK = N // NB


_SC_CHUNK = 112


def _sc_gather_body(a_hbm, xt_hbm, yt_hbm, zt_hbm, idx_hbm,
                    outa_hbm, outx_hbm, outy_hbm, outz_hbm,
                    idx_v, rows_a, bx, by, bz, xt, yt, zt,
                    sem_g, sem_o0, sem_o1):
    info = plsc.get_sparse_core_info()
    nc = info.num_cores
    wid = lax.axis_index("s") * nc + lax.axis_index("c")
    nw = nc * info.num_subcores
    per_w = E // nw
    ch = _SC_CHUNK
    nch = (per_w + ch - 1) // ch
    last = per_w - ch

    pltpu.sync_copy(xt_hbm, xt)
    pltpu.sync_copy(yt_hbm, yt)
    pltpu.sync_copy(zt_hbm, zt)

    def chunk_base(j, b):
        return wid * per_w + jnp.minimum(j * ch, last)

    def out_copies(j, b):
        base = chunk_base(j, b)
        sem = sem_o0 if b == 0 else sem_o1
        return [
            pltpu.make_async_copy(rows_a.at[b], outa_hbm.at[pl.ds(base, ch)],
                                  sem),
            pltpu.make_async_copy(bx.at[b], outx_hbm.at[pl.ds(base, ch)],
                                  sem),
            pltpu.make_async_copy(by.at[b], outy_hbm.at[pl.ds(base, ch)],
                                  sem),
            pltpu.make_async_copy(bz.at[b], outz_hbm.at[pl.ds(base, ch)],
                                  sem),
        ]

    def body(it, _):
        for b in range(2):
            j = it * 2 + b

            @pl.when(j < nch)
            def _():
                @pl.when(j >= 2)
                def _():
                    for cp in out_copies(j - 2, b):
                        cp.wait()

                base = chunk_base(j, b)
                pltpu.sync_copy(idx_hbm.at[pl.ds(base, ch)], idx_v.at[b])
                gcp = pltpu.make_async_copy(a_hbm.at[idx_v.at[b]],
                                            rows_a.at[b], sem_g)
                gcp.start()
                for s in range(ch // 16):
                    reg = idx_v[b, pl.ds(16 * s, 16)]
                    bx[b, pl.ds(16 * s, 16)] = plsc.load_gather(xt, [reg])
                    by[b, pl.ds(16 * s, 16)] = plsc.load_gather(yt, [reg])
                    bz[b, pl.ds(16 * s, 16)] = plsc.load_gather(zt, [reg])
                gcp.wait()
                for cp in out_copies(j, b):
                    cp.start()
        return 0

    lax.fori_loop(0, (nch + 1) // 2, body, 0)
    for jt in (nch - 2, nch - 1):
        for cp in out_copies(jt, jt & 1):
            cp.wait()


def _sc_gather(a_tab, xt, yt, zt, src):
    mesh = plsc.VectorSubcoreMesh(core_axis_name="c", subcore_axis_name="s")
    fn = pl.kernel(
        _sc_gather_body,
        mesh=mesh,
        compiler_params=pltpu.CompilerParams(needs_layout_passes=False),
        out_type=[
            jax.ShapeDtypeStruct((E, D), jnp.float32),
            jax.ShapeDtypeStruct((E,), jnp.float32),
            jax.ShapeDtypeStruct((E,), jnp.float32),
            jax.ShapeDtypeStruct((E,), jnp.float32),
        ],
        scratch_types=[
            pltpu.VMEM((2, _SC_CHUNK), jnp.int32),
            pltpu.VMEM((2, _SC_CHUNK, D), jnp.float32),
            pltpu.VMEM((2, _SC_CHUNK), jnp.float32),
            pltpu.VMEM((2, _SC_CHUNK), jnp.float32),
            pltpu.VMEM((2, _SC_CHUNK), jnp.float32),
            pltpu.VMEM((N,), jnp.float32),
            pltpu.VMEM((N,), jnp.float32),
            pltpu.VMEM((N,), jnp.float32),
            pltpu.SemaphoreType.DMA,
            pltpu.SemaphoreType.DMA,
            pltpu.SemaphoreType.DMA,
        ],
    )
    return fn(a_tab, xt, yt, zt, src)


def _prep_body(feat_ref, wa_ref, wb_ref, a_ref, b_ref):
    f = feat_ref[...]
    a_ref[...] = jnp.dot(f, wa_ref[...], preferred_element_type=jnp.float32)
    b_ref[...] = jnp.dot(f, wb_ref[...], preferred_element_type=jnp.float32)


def _prep(feat, wa, wb):
    return pl.pallas_call(
        _prep_body,
        grid=(NBLK,),
        in_specs=[
            pl.BlockSpec((NB, D), lambda i: (i, 0)),
            pl.BlockSpec((D, H), lambda i: (0, 0)),
            pl.BlockSpec((D, H), lambda i: (0, 0)),
        ],
        out_specs=[
            pl.BlockSpec((NB, H), lambda i: (i, 0)),
            pl.BlockSpec((NB, H), lambda i: (i, 0)),
        ],
        out_shape=[
            jax.ShapeDtypeStruct((N, H), jnp.float32),
            jax.ShapeDtypeStruct((N, H), jnp.float32),
        ],
    )(feat, wa, wb)


def _delta2d(mails, cols):
    nb = mails[0].shape[0]
    eb = nb * DEG
    acc = jnp.zeros((eb, DEG), jnp.float32)
    for mail, col in zip(mails, cols):
        mrep = jnp.broadcast_to(mail[:, None, :], (nb, DEG, DEG))
        mrep = mrep.reshape(eb, DEG)
        d = mrep - col
        acc = acc + d * d
    return acc


def _dsum_body(cxm_ref, cym_ref, czm_ref, out_ref):
    tot = jnp.zeros((), jnp.float32)
    for ref in (cxm_ref, cym_ref, czm_ref):
        m = ref[...]
        tot += 2.0 * DEG * jnp.sum(m * m)
        rs = jnp.sum(m, axis=1)
        tot -= 2.0 * jnp.sum(rs * rs)
    blk = tot.reshape(1, 1)

    @pl.when(pl.program_id(0) == 0)
    def _():
        out_ref[...] = jnp.zeros_like(out_ref)

    out_ref[...] += blk


def _dsum(cxm, cym, czm):
    return pl.pallas_call(
        _dsum_body,
        grid=(NBLK,),
        in_specs=[pl.BlockSpec((NB, 16), lambda i: (i, 0))] * 3,
        out_specs=pl.BlockSpec((1, 1), lambda i: (0, 0)),
        out_shape=jax.ShapeDtypeStruct((1, 1), jnp.float32),
    )(cxm, cym, czm)


def _silu(x):
    return 0.5 * x * (jnp.tanh(0.5 * x) + 1.0)


def _lane_tree(x, op):
    w = x.shape[1]
    while w > HS:
        half = w // 2
        x = op(x[:, :half], x[:, half:])
        w = half
    return x


def _main_body(as_ref, cxm_ref, cym_ref, czm_ref, ccol_ref, crep_ref,
               coord_ref, feat_ref, b_ref, s_ref,
               w1t_ref, b1t_ref, w2big_ref, b2t_ref,
               esw_ref, esb_ref, nsw_ref, nsb_ref,
               ew1h_ref, ew1s_ref, eb1_ref, ew2_ref, eb2_ref,
               cw1_ref, cb1_ref, cw2_ref, cb2_ref,
               nw1f_ref, nw1h_ref, nw1v_ref, nb1_ref, nw2_ref, nb2_ref,
               h_out_ref, x_out_ref):
    mails = [cxm_ref[...], cym_ref[...], czm_ref[...]]
    ccol = ccol_ref[...]
    cols = [ccol[:, k:k + 1] for k in range(3)]
    coord = coord_ref[...]

    inv = 1.0 / (s_ref[0, 0] + 1.0)
    delta = _delta2d(mails, cols) * inv

    rows16 = lax.broadcasted_iota(jnp.int32, (DEG, D), 0)
    lanes = lax.broadcasted_iota(jnp.int32, (DEG, D), 1)
    exp_mat = (lanes // HS == rows16).astype(jnp.bfloat16)
    delta_b = jnp.dot(delta.astype(jnp.bfloat16), exp_mat,
                      preferred_element_type=jnp.float32)

    h1 = _silu(delta_b * w1t_ref[...] + b1t_ref[...])
    h2 = _silu(jnp.dot(h1.astype(jnp.bfloat16), w2big_ref[...],
                       preferred_element_type=jnp.float32) + b2t_ref[...])

    lanes128 = lax.broadcasted_iota(jnp.int32, (D, HS), 0)
    ch8 = lax.broadcasted_iota(jnp.int32, (D, HS), 1)
    sum_mat = (lanes128 % HS == ch8).astype(jnp.bfloat16)
    h2b = h2.astype(jnp.bfloat16)
    s1 = jnp.dot(h2b, sum_mat, preferred_element_type=jnp.float32)
    sq1 = jnp.dot(h2b * h2b, sum_mat, preferred_element_type=jnp.float32)
    mean1 = s1 * (1.0 / DEG)
    var1 = sq1 * (1.0 / DEG) - mean1 * mean1
    std1 = jnp.sqrt(jnp.maximum(var1, 0.0))
    mx1 = _lane_tree(h2b, jnp.maximum)
    mn1 = _lane_tree(h2b, jnp.minimum)

    esw = esw_ref[...]
    eswb = esw.astype(jnp.bfloat16)
    acc = jnp.dot(s1, esw[0:8], preferred_element_type=jnp.float32)
    acc += jnp.dot(mean1, esw[8:16], preferred_element_type=jnp.float32)
    acc += jnp.dot(mx1, eswb[16:24], preferred_element_type=jnp.float32)
    acc += jnp.dot(mn1, eswb[24:32], preferred_element_type=jnp.float32)
    acc += jnp.dot(std1, esw[32:40], preferred_element_type=jnp.float32)
    h_e_dx = _silu(acc + esb_ref[...])

    nb = NB
    hr3 = h_e_dx.reshape(nb, DEG, HS)
    s2 = hr3[:, 0, :]
    sq2 = s2 * s2
    mx2 = s2
    mn2 = s2
    for i in range(1, DEG):
        v = hr3[:, i, :]
        s2 = s2 + v
        sq2 = sq2 + v * v
        mx2 = jnp.maximum(mx2, v)
        mn2 = jnp.minimum(mn2, v)
    mean2 = s2 * (1.0 / DEG)
    var2 = sq2 * (1.0 / DEG) - mean2 * mean2
    std2 = jnp.sqrt(jnp.maximum(var2, 0.0))
    nsw = nsw_ref[...]
    acc2 = jnp.dot(s2, nsw[0:8], preferred_element_type=jnp.float32)
    acc2 += jnp.dot(mean2, nsw[8:16], preferred_element_type=jnp.float32)
    acc2 += jnp.dot(mx2, nsw[16:24], preferred_element_type=jnp.float32)
    acc2 += jnp.dot(mn2, nsw[24:32], preferred_element_type=jnp.float32)
    acc2 += jnp.dot(std2, nsw[32:40], preferred_element_type=jnp.float32)
    h_v_dx = _silu(acc2 + nsb_ref[...])

    b_rep = jnp.broadcast_to(b_ref[...][:, None, :], (nb, DEG, H))
    b_rep = b_rep.reshape(EB, H)
    dcat = ccol - crep_ref[...]
    sqd = jnp.sum(dcat * dcat, axis=1, keepdims=True)
    z1 = as_ref[...] + b_rep
    z1 += jnp.dot(h_e_dx, ew1h_ref[...], preferred_element_type=jnp.float32)
    z1 += sqd * ew1s_ref[...] + eb1_ref[...]
    h1e = _silu(z1)
    h_e = _silu(jnp.dot(h1e.astype(jnp.bfloat16), ew2_ref[...],
                        preferred_element_type=jnp.float32) + eb2_ref[...])

    ch = _silu(jnp.dot(h_e.astype(jnp.bfloat16), cw1_ref[...],
                       preferred_element_type=jnp.float32) + cb1_ref[...])
    coef = jnp.dot(ch, cw2_ref[...],
                   preferred_element_type=jnp.float32) + cb2_ref[0, 0]
    g3 = (dcat * coef).reshape(nb, DEG, 3)
    xa = g3[:, 0, :]
    for i in range(1, DEG):
        xa = xa + g3[:, i, :]
    x_out_ref[...] = coord + xa

    he3 = h_e.reshape(nb, DEG, H)
    h_agg = he3[:, 0, :]
    for i in range(1, DEG):
        h_agg = h_agg + he3[:, i, :]
    z = jnp.dot(feat_ref[...].astype(jnp.bfloat16), nw1f_ref[...],
                preferred_element_type=jnp.float32)
    z += jnp.dot(h_agg.astype(jnp.bfloat16), nw1h_ref[...],
                 preferred_element_type=jnp.float32)
    z += jnp.dot(h_v_dx, nw1v_ref[...], preferred_element_type=jnp.float32)
    h_new = jnp.dot(_silu(z + nb1_ref[...]).astype(jnp.bfloat16), nw2_ref[...],
                    preferred_element_type=jnp.float32) + nb2_ref[...]
    h_out_ref[...] = h_new


def _full(x):
    return pl.BlockSpec(x, lambda i: tuple(0 for _ in x))


def _main(as_g, cxm, cym, czm, ccol, crep, coordinate, feat, b_tab,
          s_val, wp):
    in_specs = [
        pl.BlockSpec((EB, D), lambda i: (i, 0)),
        pl.BlockSpec((NB, 16), lambda i: (i, 0)),
        pl.BlockSpec((NB, 16), lambda i: (i, 0)),
        pl.BlockSpec((NB, 16), lambda i: (i, 0)),
        pl.BlockSpec((EB, 3), lambda i: (i, 0)),
        pl.BlockSpec((EB, 3), lambda i: (i, 0)),
        pl.BlockSpec((NB, 3), lambda i: (i, 0)),
        pl.BlockSpec((NB, D), lambda i: (i, 0)),
        pl.BlockSpec((NB, H), lambda i: (i, 0)),
        _full((1, 1)),
        _full((1, D)), _full((1, D)), _full((D, D)), _full((1, D)),
        _full((5 * HS, HS)), _full((1, HS)), _full((5 * HS, HS)), _full((1, HS)),
        _full((HS, H)), _full((1, H)), _full((1, H)), _full((H, H)), _full((1, H)),
        _full((H, H)), _full((1, H)), _full((H, 1)), _full((1, 1)),
        _full((D, H)), _full((H, H)), _full((HS, H)), _full((1, H)), _full((H, D)),
        _full((1, D)),
    ]
    out_specs = [
        pl.BlockSpec((NB, D), lambda i: (i, 0)),
        pl.BlockSpec((NB, 3), lambda i: (i, 0)),
    ]
    return pl.pallas_call(
        _main_body,
        grid=(NBLK,),
        in_specs=in_specs,
        out_specs=out_specs,
        out_shape=[
            jax.ShapeDtypeStruct((N, D), jnp.float32),
            jax.ShapeDtypeStruct((N, 3), jnp.float32),
        ],
    )(as_g, cxm, cym, czm, ccol, crep, coordinate, feat, b_tab,
      s_val, *wp)


def _weight_prep(p):
    bf = jnp.bfloat16
    w1t = jnp.tile(p['dW1'].reshape(1, HS), (1, DEG))
    b1t = jnp.tile(p['db1'].reshape(1, HS), (1, DEG))
    w2big = jnp.kron(jnp.eye(DEG, dtype=jnp.float32), p['dW2'])
    b2t = jnp.tile(p['db2'].reshape(1, HS), (1, DEG))
    return [
        w1t, b1t, w2big.astype(bf), b2t,
        p['esW'], p['esb'].reshape(1, HS), p['nsW'], p['nsb'].reshape(1, HS),
        p['eW1'][0:HS], p['eW1'][HS + 2 * D:HS + 2 * D + 1],
        p['eb1'].reshape(1, H), p['eW2'].astype(bf), p['eb2'].reshape(1, H),
        p['cW1'].astype(bf), p['cb1'].reshape(1, H), p['cW2'],
        p['cb2'].reshape(1, 1),
        p['nW1'][0:D].astype(bf), p['nW1'][D:D + H].astype(bf),
        p['nW1'][D + H:D + H + HS],
        p['nb1'].reshape(1, H), p['nW2'].astype(bf), p['nb2'].reshape(1, D),
    ]


@jax.jit
def kernel(feat, coordinate, edge_index, params):
    p = params
    src = edge_index[0].astype(jnp.int32)
    wa = p['eW1'][HS:HS + D]
    wb = p['eW1'][HS + D:HS + 2 * D]
    a_tab, b_tab = _prep(feat, wa, wb)
    as_g, cx, cy, cz = _sc_gather(
        a_tab, coordinate[:, 0], coordinate[:, 1], coordinate[:, 2], src)
    cxm = cx.reshape(N, DEG)
    cym = cy.reshape(N, DEG)
    czm = cz.reshape(N, DEG)
    s_val = _dsum(cxm, cym, czm)
    wp = _weight_prep(p)
    ccol = jnp.stack([cx, cy, cz], axis=1)
    crep = jnp.repeat(coordinate, DEG, axis=0)
    h_new, x_new = _main(as_g, cxm, cym, czm, ccol, crep,
                         coordinate, feat, b_tab, s_val, wp)
    return h_new, x_new
